# Initial kernel scaffold; baseline (speedup 1.0000x reference)
import jax
import jax.numpy as jnp
from jax.experimental import pallas as pl


def kernel(x, edge_index, batch, emb_table, W1, b1, W2, b2, Wfc, bfc):
    raise NotImplementedError("write your pallas kernel here")


# speedup vs baseline: 14.2017x; 14.2017x over previous
import functools

import jax
import jax.numpy as jnp
from jax import lax
from jax.experimental import pallas as pl
from jax.experimental.pallas import tpu as pltpu
from jax.experimental.pallas import tpu_sc as plsc

N = 10000
E = 320000
V = 30522
S = 8
D = 768
H = 128
NG = 128
NCLS = 8

NC = 2
NS = 16
NW = NC * NS
NPAD = 10240
TPN = NPAD // NW
RPT = NPAD // NS
EPW = E // NW
CE = 200
CN = 64


def _proj_body(t_ref, w_ref, o_ref):
    o_ref[...] = jnp.dot(t_ref[...], w_ref[...],
                         preferred_element_type=jnp.float32)


def _proj(tbl, w1):
    bm = 1536
    return pl.pallas_call(
        _proj_body,
        grid=(pl.cdiv(V, bm),),
        in_specs=[pl.BlockSpec((bm, D), lambda i: (i, 0)),
                  pl.BlockSpec((D, H), lambda i: (0, 0))],
        out_specs=pl.BlockSpec((bm, H), lambda i: (i, 0)),
        out_shape=jax.ShapeDtypeStruct((V, H), jnp.float32),
    )(tbl, w1)


def _dinv_body(deg_ref, o_ref):
    d = deg_ref[0] + deg_ref[1] + 1.0
    o_ref[...] = lax.rsqrt(jnp.maximum(d, 1.0))


def _dinv(deg):
    return pl.pallas_call(
        _dinv_body,
        out_shape=jax.ShapeDtypeStruct((NPAD, 16), jnp.float32),
    )(deg)


def _mm2_body(acc_ref, hws_ref, dv_ref, w2_ref, b1_ref, o_ref):
    dv = dv_ref[:, :1]
    h1 = jnp.maximum((acc_ref[0] + acc_ref[1] + hws_ref[...]) * dv
                     + b1_ref[...], 0.0)
    o_ref[...] = jnp.dot(h1, w2_ref[...],
                         preferred_element_type=jnp.float32) * dv


def _mm2(parts, hws, dinv16, w2, b1r):
    bm = 2048
    return pl.pallas_call(
        _mm2_body,
        grid=(NPAD // bm,),
        in_specs=[pl.BlockSpec((2, bm, H), lambda i: (0, i, 0)),
                  pl.BlockSpec((bm, H), lambda i: (i, 0)),
                  pl.BlockSpec((bm, 16), lambda i: (i, 0)),
                  pl.BlockSpec((H, H), lambda i: (0, 0)),
                  pl.BlockSpec((1, H), lambda i: (0, 0))],
        out_specs=pl.BlockSpec((bm, H), lambda i: (i, 0)),
        out_shape=jax.ShapeDtypeStruct((NPAD, H), jnp.float32),
    )(parts, hws, dinv16, w2, b1r)


def _pool_body(acc_ref, hws_ref, dv_ref, b_ref, b2_ref, wfc_ref, bfc_ref,
               o_ref, g_ref, cnt_ref):
    k = pl.program_id(0)

    @pl.when(k == 0)
    def _():
        g_ref[...] = jnp.zeros_like(g_ref)
        cnt_ref[...] = jnp.zeros_like(cnt_ref)

    ids = b_ref[...]
    eq = (ids[None, :] == lax.broadcasted_iota(jnp.int32, (NG, ids.shape[0]),
                                               0)).astype(jnp.float32)
    h = (acc_ref[0] + acc_ref[1] + hws_ref[...]) * dv_ref[:, :1]
    g_ref[...] += jnp.dot(eq, h, preferred_element_type=jnp.float32)
    cnt_ref[...] += jnp.sum(eq, axis=1, keepdims=True)

    @pl.when(k == pl.num_programs(0) - 1)
    def _():
        cnt = cnt_ref[...]
        g = (g_ref[...] / jnp.maximum(cnt, 1.0)
             + b2_ref[...] * (cnt > 0.0).astype(jnp.float32))
        o_ref[...] = jnp.dot(g, wfc_ref[...],
                             preferred_element_type=jnp.float32) + bfc_ref[...]


def _pool(parts, hws, dinv16, batp, b2r, wfc, bfcr):
    bk = 2048
    return pl.pallas_call(
        _pool_body,
        grid=(NPAD // bk,),
        in_specs=[pl.BlockSpec((2, bk, H), lambda i: (0, i, 0)),
                  pl.BlockSpec((bk, H), lambda i: (i, 0)),
                  pl.BlockSpec((bk, 16), lambda i: (i, 0)),
                  pl.BlockSpec((bk,), lambda i: (i,)),
                  pl.BlockSpec((1, H), lambda i: (0, 0)),
                  pl.BlockSpec((H, NCLS), lambda i: (0, 0)),
                  pl.BlockSpec((1, NCLS), lambda i: (0, 0))],
        out_specs=pl.BlockSpec((NG, NCLS), lambda i: (0, 0)),
        out_shape=jax.ShapeDtypeStruct((NG, NCLS), jnp.float32),
        scratch_shapes=[pltpu.VMEM((NG, H), jnp.float32),
                        pltpu.VMEM((NG, 1), jnp.float32)],
    )(parts, hws, dinv16, batp, b2r, wfc, bfcr)


@functools.cache
def _sc_kernels():
    mesh = plsc.VectorSubcoreMesh(core_axis_name="c", subcore_axis_name="s",
                                  num_cores=NC, num_subcores=NS)

    @functools.partial(
        pl.kernel,
        compiler_params=pltpu.CompilerParams(use_tc_tiling_on_sc=False),
        out_type=jax.ShapeDtypeStruct((NC, NPAD, 16), jnp.float32),
        mesh=mesh,
        scratch_types=[
            pltpu.VMEM((CE,), jnp.int32),
            pltpu.VMEM((CE, 16), jnp.float32),
            pltpu.VMEM((RPT, 16), jnp.float32),
            pltpu.VMEM_SHARED((NPAD, 16), jnp.float32),
        ],
    )
    def _deg(dst_hbm, out_hbm, di_v, ones_v, zb_v, acc_sh):
        c = lax.axis_index("c")
        s = lax.axis_index("s")
        wid = s * NC + c

        @pl.loop(0, CE)
        def _fill(r):
            ones_v[r] = jnp.ones((16,), jnp.float32)

        @pl.loop(0, RPT)
        def _zero(r):
            zb_v[r] = jnp.zeros((16,), jnp.float32)

        pltpu.sync_copy(zb_v, acc_sh.at[pl.ds(s * RPT, RPT)])
        plsc.subcore_barrier()

        @pl.loop(0, EPW // CE)
        def _chunk(i):
            pltpu.sync_copy(dst_hbm.at[pl.ds(wid * EPW + i * CE, CE)], di_v)
            pltpu.sync_copy(ones_v, acc_sh.at[di_v], add=True)

        plsc.subcore_barrier()
        pltpu.sync_copy(acc_sh.at[pl.ds(s * RPT, RPT)],
                        out_hbm.at[c, pl.ds(s * RPT, RPT)])

    @functools.partial(
        pl.kernel,
        compiler_params=pltpu.CompilerParams(use_tc_tiling_on_sc=False),
        out_type=jax.ShapeDtypeStruct((NPAD, H), jnp.float32),
        mesh=mesh,
        scratch_types=[
            pltpu.VMEM((CN * S,), jnp.int32),
            pltpu.VMEM((CN * S, H), jnp.float32),
            pltpu.VMEM((CN, H), jnp.float32),
            pltpu.VMEM((TPN, 16), jnp.float32),
            pltpu.SemaphoreType.DMA,
        ],
    )
    def _tok(p_hbm, xf_hbm, dinv_hbm, out_hbm, idx_v, rows_v, ob_v, dv_v, sem):
        c = lax.axis_index("c")
        s = lax.axis_index("s")
        wid = s * NC + c
        nbase = wid * TPN
        pltpu.sync_copy(dinv_hbm.at[pl.ds(nbase, TPN)], dv_v)

        @pl.loop(0, TPN // CN)
        def _chunk(cc):
            nb = nbase + cc * CN
            pltpu.sync_copy(xf_hbm.at[pl.ds(nb * S, CN * S)], idx_v)
            pltpu.async_copy(p_hbm.at[idx_v], rows_v, sem).wait()

            @pl.loop(0, CN)
            def _node(j):
                dvvec = dv_v[cc * CN + j, pl.ds(0, 16)]
                sc = dvvec[0] * (1.0 / S)
                for k in range(H // 16):
                    acc = rows_v[S * j, pl.ds(16 * k, 16)]
                    for t in range(1, S):
                        acc = acc + rows_v[S * j + t, pl.ds(16 * k, 16)]
                    ob_v[j, pl.ds(16 * k, 16)] = acc * sc

            pltpu.sync_copy(ob_v, out_hbm.at[pl.ds(nb, CN)])

    @functools.partial(
        pl.kernel,
        compiler_params=pltpu.CompilerParams(use_tc_tiling_on_sc=False),
        out_type=jax.ShapeDtypeStruct((NC, NPAD, H), jnp.float32),
        mesh=mesh,
        scratch_types=[
            pltpu.VMEM((CE,), jnp.int32),
            pltpu.VMEM((CE,), jnp.int32),
            pltpu.VMEM((CE, H), jnp.float32),
            pltpu.VMEM((RPT // 8, H), jnp.float32),
            pltpu.VMEM_SHARED((NPAD, H), jnp.float32),
            pltpu.SemaphoreType.DMA,
        ],
    )
    def _msg(hws_hbm, src_hbm, dst_hbm, out_hbm, si_v, di_v, rows_v, zb_v,
             acc_sh, sem):
        c = lax.axis_index("c")
        s = lax.axis_index("s")
        wid = s * NC + c

        @pl.loop(0, RPT // 8)
        def _zero(r):
            for k in range(H // 16):
                zb_v[r, pl.ds(16 * k, 16)] = jnp.zeros((16,), jnp.float32)

        @pl.loop(0, 8)
        def _zs(i):
            pltpu.sync_copy(zb_v, acc_sh.at[pl.ds(s * RPT + i * (RPT // 8),
                                                  RPT // 8)])

        plsc.subcore_barrier()

        @pl.loop(0, EPW // CE)
        def _chunk(i):
            base = wid * EPW + i * CE
            pltpu.sync_copy(src_hbm.at[pl.ds(base, CE)], si_v)
            pltpu.sync_copy(dst_hbm.at[pl.ds(base, CE)], di_v)
            pltpu.async_copy(hws_hbm.at[si_v], rows_v, sem).wait()
            pltpu.sync_copy(rows_v, acc_sh.at[di_v], add=True)

        plsc.subcore_barrier()
        pltpu.sync_copy(acc_sh.at[pl.ds(s * RPT, RPT)],
                        out_hbm.at[c, pl.ds(s * RPT, RPT)])

    return _deg, _tok, _msg


def kernel(x, edge_index, batch, emb_table, W1, b1, W2, b2, Wfc, bfc):
    x = x.astype(jnp.int32)
    ei = edge_index.astype(jnp.int32)
    bat = batch.astype(jnp.int32)

    xf = jnp.pad(x.reshape(-1), (0, NPAD * S - N * S))
    src = ei[0]
    dst = ei[1]
    batp = jnp.pad(bat, (0, NPAD - N), constant_values=-1)

    _deg, _tok, _msg = _sc_kernels()
    p = _proj(emb_table, W1)
    deg = _deg(dst)
    dinv16 = _dinv(deg)
    hw1s = _tok(p, xf, dinv16)
    parts1 = _msg(hw1s, src, dst)
    hw2s = _mm2(parts1, hw1s, dinv16, W2, b1.reshape(1, H))
    parts2 = _msg(hw2s, src, dst)
    return _pool(parts2, hw2s, dinv16, batp, b2.reshape(1, H), Wfc,
                 bfc.reshape(1, NCLS))


# speedup vs baseline: 18.3173x; 1.2898x over previous
import functools

import jax
import jax.numpy as jnp
from jax import lax
from jax.experimental import pallas as pl
from jax.experimental.pallas import tpu as pltpu
from jax.experimental.pallas import tpu_sc as plsc

N = 10000
E = 320000
V = 30522
S = 8
D = 768
H = 128
HH = H // 2
NG = 128
NCLS = 8

NC = 2
NS = 16
NW = NC * NS
NPAD = 10240
RPT = NPAD // NS
EPW = E // NW
CED = 200
CEM = 200
CET = 256
TOK = NPAD * S


def _proj_body(t_ref, w_ref, o_ref):
    r = jnp.dot(t_ref[...], w_ref[...], preferred_element_type=jnp.float32)
    o_ref[0] = r[:, :HH]
    o_ref[1] = r[:, HH:]


def _proj(tbl, w1):
    bm = 1536
    return pl.pallas_call(
        _proj_body,
        grid=(pl.cdiv(V, bm),),
        in_specs=[pl.BlockSpec((bm, D), lambda i: (i, 0)),
                  pl.BlockSpec((D, H), lambda i: (0, 0))],
        out_specs=pl.BlockSpec((NC, bm, HH), lambda i: (0, i, 0)),
        out_shape=jax.ShapeDtypeStruct((NC, V, HH), jnp.float32),
    )(tbl, w1)


def _dinv_body(deg_ref, o_ref):
    d = deg_ref[0] + deg_ref[1] + 1.0
    o_ref[...] = lax.rsqrt(jnp.maximum(d, 1.0))


def _dinv(deg):
    return pl.pallas_call(
        _dinv_body,
        out_shape=jax.ShapeDtypeStruct((NPAD, 16), jnp.float32),
    )(deg)


def _scale_body(t_ref, dv_ref, o_ref):
    o_ref[0] = t_ref[0] * (dv_ref[:, :1] * (1.0 / S))


def _scale(tokacc, dinv16):
    bm = 2048
    return pl.pallas_call(
        _scale_body,
        grid=(NPAD // bm, NC),
        in_specs=[pl.BlockSpec((1, bm, HH), lambda i, c: (c, i, 0)),
                  pl.BlockSpec((bm, 16), lambda i, c: (i, 0))],
        out_specs=pl.BlockSpec((1, bm, HH), lambda i, c: (c, i, 0)),
        out_shape=jax.ShapeDtypeStruct((NC, NPAD, HH), jnp.float32),
    )(tokacc, dinv16)


def _mm2_body(acc_ref, hws_ref, dv_ref, w2_ref, b1_ref, o_ref):
    dv = dv_ref[:, :1]
    a = jnp.concatenate([acc_ref[0] + hws_ref[0],
                         acc_ref[1] + hws_ref[1]], axis=1)
    h1 = jnp.maximum(a * dv + b1_ref[...], 0.0)
    r = jnp.dot(h1, w2_ref[...], preferred_element_type=jnp.float32) * dv
    o_ref[0] = r[:, :HH]
    o_ref[1] = r[:, HH:]


def _mm2(parts, hws, dinv16, w2, b1r):
    bm = 2048
    return pl.pallas_call(
        _mm2_body,
        grid=(NPAD // bm,),
        in_specs=[pl.BlockSpec((NC, bm, HH), lambda i: (0, i, 0)),
                  pl.BlockSpec((NC, bm, HH), lambda i: (0, i, 0)),
                  pl.BlockSpec((bm, 16), lambda i: (i, 0)),
                  pl.BlockSpec((H, H), lambda i: (0, 0)),
                  pl.BlockSpec((1, H), lambda i: (0, 0))],
        out_specs=pl.BlockSpec((NC, bm, HH), lambda i: (0, i, 0)),
        out_shape=jax.ShapeDtypeStruct((NC, NPAD, HH), jnp.float32),
    )(parts, hws, dinv16, w2, b1r)


def _pool_body(acc_ref, hws_ref, dv_ref, b_ref, b2_ref, wfc_ref, bfc_ref,
               o_ref, g_ref, cnt_ref):
    k = pl.program_id(0)

    @pl.when(k == 0)
    def _():
        g_ref[...] = jnp.zeros_like(g_ref)
        cnt_ref[...] = jnp.zeros_like(cnt_ref)

    ids = b_ref[...]
    eq = (ids[None, :] == lax.broadcasted_iota(jnp.int32, (NG, ids.shape[0]),
                                               0)).astype(jnp.float32)
    h = jnp.concatenate([acc_ref[0] + hws_ref[0],
                         acc_ref[1] + hws_ref[1]], axis=1) * dv_ref[:, :1]
    g_ref[...] += jnp.dot(eq, h, preferred_element_type=jnp.float32)
    cnt_ref[...] += jnp.sum(eq, axis=1, keepdims=True)

    @pl.when(k == pl.num_programs(0) - 1)
    def _():
        cnt = cnt_ref[...]
        g = (g_ref[...] / jnp.maximum(cnt, 1.0)
             + b2_ref[...] * (cnt > 0.0).astype(jnp.float32))
        o_ref[...] = jnp.dot(g, wfc_ref[...],
                             preferred_element_type=jnp.float32) + bfc_ref[...]


def _pool(parts, hws, dinv16, batp, b2r, wfc, bfcr):
    bk = 2048
    return pl.pallas_call(
        _pool_body,
        grid=(NPAD // bk,),
        in_specs=[pl.BlockSpec((NC, bk, HH), lambda i: (0, i, 0)),
                  pl.BlockSpec((NC, bk, HH), lambda i: (0, i, 0)),
                  pl.BlockSpec((bk, 16), lambda i: (i, 0)),
                  pl.BlockSpec((bk,), lambda i: (i,)),
                  pl.BlockSpec((1, H), lambda i: (0, 0)),
                  pl.BlockSpec((H, NCLS), lambda i: (0, 0)),
                  pl.BlockSpec((1, NCLS), lambda i: (0, 0))],
        out_specs=pl.BlockSpec((NG, NCLS), lambda i: (0, 0)),
        out_shape=jax.ShapeDtypeStruct((NG, NCLS), jnp.float32),
        scratch_shapes=[pltpu.VMEM((NG, H), jnp.float32),
                        pltpu.VMEM((NG, 1), jnp.float32)],
    )(parts, hws, dinv16, batp, b2r, wfc, bfcr)


@functools.cache
def _sc_kernels():
    mesh = plsc.VectorSubcoreMesh(core_axis_name="c", subcore_axis_name="s",
                                  num_cores=NC, num_subcores=NS)

    @functools.partial(
        pl.kernel,
        compiler_params=pltpu.CompilerParams(use_tc_tiling_on_sc=False),
        out_type=jax.ShapeDtypeStruct((NC, NPAD, 16), jnp.float32),
        mesh=mesh,
        scratch_types=[
            pltpu.VMEM((CED,), jnp.int32),
            pltpu.VMEM((CED, 16), jnp.float32),
            pltpu.VMEM((RPT, 16), jnp.float32),
            pltpu.VMEM_SHARED((NPAD, 16), jnp.float32),
        ],
    )
    def _deg(dst_hbm, out_hbm, di_v, ones_v, zb_v, acc_sh):
        c = lax.axis_index("c")
        s = lax.axis_index("s")
        wid = s * NC + c

        @pl.loop(0, CED)
        def _fill(r):
            ones_v[r] = jnp.ones((16,), jnp.float32)

        @pl.loop(0, RPT)
        def _zero(r):
            zb_v[r] = jnp.zeros((16,), jnp.float32)

        pltpu.sync_copy(zb_v, acc_sh.at[pl.ds(s * RPT, RPT)])
        plsc.subcore_barrier()

        @pl.loop(0, EPW // CED)
        def _chunk(i):
            pltpu.sync_copy(dst_hbm.at[pl.ds(wid * EPW + i * CED, CED)], di_v)
            pltpu.sync_copy(ones_v, acc_sh.at[di_v], add=True)

        plsc.subcore_barrier()
        pltpu.sync_copy(acc_sh.at[pl.ds(s * RPT, RPT)],
                        out_hbm.at[c, pl.ds(s * RPT, RPT)])

    def _make_scat(tot, ce, name):
        nch = tot // NS // ce
        assert tot % (NS * ce) == 0 and nch % 2 == 0 and ce % 8 == 0
        zsegs = []
        off = 0
        while off < RPT:
            step = min(ce, RPT - off)
            zsegs.append((off, step))
            off += step

        @functools.partial(
            pl.kernel,
            compiler_params=pltpu.CompilerParams(use_tc_tiling_on_sc=False),
            out_type=jax.ShapeDtypeStruct((NC, NPAD, HH), jnp.float32),
            mesh=mesh,
            scratch_types=[
                pltpu.VMEM((nch, ce), jnp.int32),
                pltpu.VMEM((nch, ce), jnp.int32),
                pltpu.VMEM((ce, HH), jnp.float32),
                pltpu.VMEM((ce, HH), jnp.float32),
                pltpu.VMEM_SHARED((NPAD, HH), jnp.float32),
                pltpu.SemaphoreType.DMA,
                pltpu.SemaphoreType.DMA,
            ],
            name=name,
        )
        def _scat(tbl_hbm, si2_hbm, di2_hbm, out_hbm, si_v, di_v,
                  rows0_v, rows1_v, acc_sh, sem0, sem1):
            c = lax.axis_index("c")
            s = lax.axis_index("s")

            pltpu.sync_copy(si2_hbm.at[pl.ds(s * nch, nch)], si_v)
            pltpu.sync_copy(di2_hbm.at[pl.ds(s * nch, nch)], di_v)

            @pl.loop(0, ce)
            def _z(r):
                for k in range(HH // 16):
                    rows0_v[r, pl.ds(16 * k, 16)] = jnp.zeros((16,),
                                                              jnp.float32)

            for off, step in zsegs:
                pltpu.sync_copy(rows0_v.at[pl.ds(0, step)],
                                acc_sh.at[pl.ds(s * RPT + off, step)])
            plsc.subcore_barrier()

            tbl_c = tbl_hbm.at[c]
            pltpu.async_copy(tbl_c.at[si_v.at[0]], rows0_v, sem0)

            @pl.loop(0, nch // 2)
            def _chunk(i2):
                i = i2 * 2
                pltpu.async_copy(tbl_c.at[si_v.at[i + 1]], rows1_v, sem1)
                pltpu.make_async_copy(tbl_c.at[si_v.at[i]], rows0_v,
                                      sem0).wait()
                pltpu.sync_copy(rows0_v, acc_sh.at[di_v.at[i]], add=True)

                @pl.when(i + 2 < nch)
                def _():
                    pltpu.async_copy(tbl_c.at[si_v.at[i + 2]], rows0_v, sem0)

                pltpu.make_async_copy(tbl_c.at[si_v.at[i + 1]], rows1_v,
                                      sem1).wait()
                pltpu.sync_copy(rows1_v, acc_sh.at[di_v.at[i + 1]], add=True)

            plsc.subcore_barrier()
            pltpu.sync_copy(acc_sh.at[pl.ds(s * RPT, RPT)],
                            out_hbm.at[c, pl.ds(s * RPT, RPT)])

        return _scat

    return (_deg, _make_scat(TOK, CET, "sc_tok_scatter"),
            _make_scat(E, CEM, "sc_msg_scatter"))


def kernel(x, edge_index, batch, emb_table, W1, b1, W2, b2, Wfc, bfc):
    x = x.astype(jnp.int32)
    ei = edge_index.astype(jnp.int32)
    bat = batch.astype(jnp.int32)

    xf2 = jnp.pad(x.reshape(-1), (0, TOK - N * S)).reshape(TOK // CET, CET)
    nrep2 = (jnp.arange(TOK, dtype=jnp.int32) // S).reshape(TOK // CET, CET)
    src2 = ei[0].reshape(E // CEM, CEM)
    dst2 = ei[1].reshape(E // CEM, CEM)
    batp = jnp.pad(bat, (0, NPAD - N), constant_values=-1)

    _deg, _scat_tok, _scat_msg = _sc_kernels()
    p = _proj(emb_table, W1)
    deg = _deg(ei[1])
    dinv16 = _dinv(deg)
    tokacc = _scat_tok(p, xf2, nrep2)
    hw1s = _scale(tokacc, dinv16)
    parts1 = _scat_msg(hw1s, src2, dst2)
    hw2s = _mm2(parts1, hw1s, dinv16, W2, b1.reshape(1, H))
    parts2 = _scat_msg(hw2s, src2, dst2)
    return _pool(parts2, hw2s, dinv16, batp, b2.reshape(1, H), Wfc,
                 bfc.reshape(1, NCLS))


# speedup vs baseline: 21.8718x; 1.1941x over previous
import functools

import jax
import jax.numpy as jnp
from jax import lax
from jax.experimental import pallas as pl
from jax.experimental.pallas import tpu as pltpu
from jax.experimental.pallas import tpu_sc as plsc

N = 10000
E = 320000
V = 30522
S = 8
D = 768
H = 128
NG = 128
NCLS = 8

NC = 2
NS = 16
NW = NC * NS
NPAD = 10240
RPT = NPAD // NS
EPW = E // NW
CED = 200
CEM = 100
CET = 128
TOK = NPAD * S


def _proj_body(t_ref, w_ref, o_ref):
    o_ref[...] = jnp.dot(t_ref[...], w_ref[...],
                         preferred_element_type=jnp.float32)


def _proj(tbl, w1):
    bm = 1536
    return pl.pallas_call(
        _proj_body,
        grid=(pl.cdiv(V, bm),),
        in_specs=[pl.BlockSpec((bm, D), lambda i: (i, 0)),
                  pl.BlockSpec((D, H), lambda i: (0, 0))],
        out_specs=pl.BlockSpec((bm, H), lambda i: (i, 0)),
        out_shape=jax.ShapeDtypeStruct((V, H), jnp.float32),
    )(tbl, w1)


def _prep_body(deg_ref, t_ref, dv_ref, o_ref):
    d = deg_ref[0] + deg_ref[1] + 1.0
    dv = lax.rsqrt(jnp.maximum(d, 1.0))
    dv_ref[...] = dv
    o_ref[...] = (t_ref[0] + t_ref[1]) * (dv[:, :1] * (1.0 / S))


def _prep(deg, tokacc):
    bm = 2048
    return pl.pallas_call(
        _prep_body,
        grid=(NPAD // bm,),
        in_specs=[pl.BlockSpec((NC, bm, 16), lambda i: (0, i, 0)),
                  pl.BlockSpec((NC, bm, H), lambda i: (0, i, 0))],
        out_specs=[pl.BlockSpec((bm, 16), lambda i: (i, 0)),
                   pl.BlockSpec((bm, H), lambda i: (i, 0))],
        out_shape=(jax.ShapeDtypeStruct((NPAD, 16), jnp.float32),
                   jax.ShapeDtypeStruct((NPAD, H), jnp.float32)),
    )(deg, tokacc)


def _mm2_body(acc_ref, hws_ref, dv_ref, w2_ref, b1_ref, o_ref):
    dv = dv_ref[:, :1]
    a = acc_ref[0] + acc_ref[1] + hws_ref[...]
    h1 = jnp.maximum(a * dv + b1_ref[...], 0.0)
    o_ref[...] = jnp.dot(h1, w2_ref[...],
                         preferred_element_type=jnp.float32) * dv


def _mm2(parts, hws, dinv16, w2, b1r):
    bm = 2048
    return pl.pallas_call(
        _mm2_body,
        grid=(NPAD // bm,),
        in_specs=[pl.BlockSpec((NC, bm, H), lambda i: (0, i, 0)),
                  pl.BlockSpec((bm, H), lambda i: (i, 0)),
                  pl.BlockSpec((bm, 16), lambda i: (i, 0)),
                  pl.BlockSpec((H, H), lambda i: (0, 0)),
                  pl.BlockSpec((1, H), lambda i: (0, 0))],
        out_specs=pl.BlockSpec((bm, H), lambda i: (i, 0)),
        out_shape=jax.ShapeDtypeStruct((NPAD, H), jnp.float32),
    )(parts, hws, dinv16, w2, b1r)


def _pool_body(acc_ref, hws_ref, dv_ref, b_ref, b2_ref, wfc_ref, bfc_ref,
               o_ref, g_ref, cnt_ref):
    k = pl.program_id(0)

    @pl.when(k == 0)
    def _():
        g_ref[...] = jnp.zeros_like(g_ref)
        cnt_ref[...] = jnp.zeros_like(cnt_ref)

    ids = b_ref[...]
    eq = (ids[None, :] == lax.broadcasted_iota(jnp.int32, (NG, ids.shape[0]),
                                               0)).astype(jnp.float32)
    h = (acc_ref[0] + acc_ref[1] + hws_ref[...]) * dv_ref[:, :1]
    g_ref[...] += jnp.dot(eq, h, preferred_element_type=jnp.float32)
    cnt_ref[...] += jnp.sum(eq, axis=1, keepdims=True)

    @pl.when(k == pl.num_programs(0) - 1)
    def _():
        cnt = cnt_ref[...]
        g = (g_ref[...] / jnp.maximum(cnt, 1.0)
             + b2_ref[...] * (cnt > 0.0).astype(jnp.float32))
        o_ref[...] = jnp.dot(g, wfc_ref[...],
                             preferred_element_type=jnp.float32) + bfc_ref[...]


def _pool(parts, hws, dinv16, batp, b2r, wfc, bfcr):
    bk = 2048
    return pl.pallas_call(
        _pool_body,
        grid=(NPAD // bk,),
        in_specs=[pl.BlockSpec((NC, bk, H), lambda i: (0, i, 0)),
                  pl.BlockSpec((bk, H), lambda i: (i, 0)),
                  pl.BlockSpec((bk, 16), lambda i: (i, 0)),
                  pl.BlockSpec((bk,), lambda i: (i,)),
                  pl.BlockSpec((1, H), lambda i: (0, 0)),
                  pl.BlockSpec((H, NCLS), lambda i: (0, 0)),
                  pl.BlockSpec((1, NCLS), lambda i: (0, 0))],
        out_specs=pl.BlockSpec((NG, NCLS), lambda i: (0, 0)),
        out_shape=jax.ShapeDtypeStruct((NG, NCLS), jnp.float32),
        scratch_shapes=[pltpu.VMEM((NG, H), jnp.float32),
                        pltpu.VMEM((NG, 1), jnp.float32)],
    )(parts, hws, dinv16, batp, b2r, wfc, bfcr)


@functools.cache
def _sc_kernels():
    mesh = plsc.VectorSubcoreMesh(core_axis_name="c", subcore_axis_name="s",
                                  num_cores=NC, num_subcores=NS)

    ndch = EPW // CED

    @functools.partial(
        pl.kernel,
        compiler_params=pltpu.CompilerParams(use_tc_tiling_on_sc=False),
        out_type=jax.ShapeDtypeStruct((NC, NPAD, 16), jnp.float32),
        mesh=mesh,
        scratch_types=[
            pltpu.VMEM((ndch, CED), jnp.int32),
            pltpu.VMEM((CED, 16), jnp.float32),
            pltpu.VMEM((RPT, 16), jnp.float32),
            pltpu.VMEM_SHARED((NPAD, 16), jnp.float32),
            pltpu.SemaphoreType.DMA,
        ],
    )
    def _deg(dst2_hbm, out_hbm, di_v, ones_v, zb_v, acc_sh, semd):
        c = lax.axis_index("c")
        s = lax.axis_index("s")
        wid = s * NC + c

        pltpu.sync_copy(dst2_hbm.at[pl.ds(wid * ndch, ndch)], di_v)

        @pl.loop(0, CED)
        def _fill(r):
            ones_v[r] = jnp.ones((16,), jnp.float32)

        @pl.loop(0, RPT)
        def _zero(r):
            zb_v[r] = jnp.zeros((16,), jnp.float32)

        pltpu.sync_copy(zb_v, acc_sh.at[pl.ds(s * RPT, RPT)])
        plsc.subcore_barrier()

        @pl.loop(0, ndch)
        def _chunk(i):
            pltpu.async_copy(ones_v, acc_sh.at[di_v.at[i]], semd, add=True)

        @pl.loop(0, ndch)
        def _drain(i):
            pltpu.make_async_copy(ones_v, acc_sh.at[di_v.at[i]], semd).wait()

        plsc.subcore_barrier()
        pltpu.sync_copy(acc_sh.at[pl.ds(s * RPT, RPT)],
                        out_hbm.at[c, pl.ds(s * RPT, RPT)])

    def _make_scat(tot, ce, name):
        nch = tot // NW // ce
        assert tot % (NW * ce) == 0 and nch % 2 == 0
        zsegs = []
        off = 0
        while off < RPT:
            step = min(ce, RPT - off)
            zsegs.append((off, step))
            off += step

        @functools.partial(
            pl.kernel,
            compiler_params=pltpu.CompilerParams(use_tc_tiling_on_sc=False),
            out_type=jax.ShapeDtypeStruct((NC, NPAD, H), jnp.float32),
            mesh=mesh,
            scratch_types=[
                pltpu.VMEM((nch, ce), jnp.int32),
                pltpu.VMEM((nch, ce), jnp.int32),
                pltpu.VMEM((ce, H), jnp.float32),
                pltpu.VMEM((ce, H), jnp.float32),
                pltpu.VMEM_SHARED((NPAD, H), jnp.float32),
                pltpu.SemaphoreType.DMA,
                pltpu.SemaphoreType.DMA,
            ],
            name=name,
        )
        def _scat(tbl_hbm, si2_hbm, di2_hbm, out_hbm, si_v, di_v,
                  rows0_v, rows1_v, acc_sh, sem0, sem1):
            c = lax.axis_index("c")
            s = lax.axis_index("s")
            wid = s * NC + c

            pltpu.sync_copy(si2_hbm.at[pl.ds(wid * nch, nch)], si_v)
            pltpu.sync_copy(di2_hbm.at[pl.ds(wid * nch, nch)], di_v)

            @pl.loop(0, ce)
            def _z(r):
                for k in range(H // 16):
                    rows0_v[r, pl.ds(16 * k, 16)] = jnp.zeros((16,),
                                                              jnp.float32)

            for off, step in zsegs:
                pltpu.sync_copy(rows0_v.at[pl.ds(0, step)],
                                acc_sh.at[pl.ds(s * RPT + off, step)])
            plsc.subcore_barrier()

            pltpu.async_copy(tbl_hbm.at[si_v.at[0]], rows0_v, sem0)
            pltpu.async_copy(tbl_hbm.at[si_v.at[1]], rows1_v, sem1)

            @pl.loop(0, nch // 2)
            def _chunk(i2):
                i = i2 * 2
                pltpu.make_async_copy(tbl_hbm.at[si_v.at[i]], rows0_v,
                                      sem0).wait()
                pltpu.sync_copy(rows0_v, acc_sh.at[di_v.at[i]], add=True)

                @pl.when(i + 2 < nch)
                def _():
                    pltpu.async_copy(tbl_hbm.at[si_v.at[i + 2]], rows0_v, sem0)

                pltpu.make_async_copy(tbl_hbm.at[si_v.at[i + 1]], rows1_v,
                                      sem1).wait()
                pltpu.sync_copy(rows1_v, acc_sh.at[di_v.at[i + 1]], add=True)

                @pl.when(i + 3 < nch)
                def _():
                    pltpu.async_copy(tbl_hbm.at[si_v.at[i + 3]], rows1_v, sem1)

            plsc.subcore_barrier()
            pltpu.sync_copy(acc_sh.at[pl.ds(s * RPT, RPT)],
                            out_hbm.at[c, pl.ds(s * RPT, RPT)])

        return _scat

    return (_deg, _make_scat(TOK, CET, "sc_tok_scatter"),
            _make_scat(E, CEM, "sc_msg_scatter"))


def kernel(x, edge_index, batch, emb_table, W1, b1, W2, b2, Wfc, bfc):
    x = x.astype(jnp.int32)
    ei = edge_index.astype(jnp.int32)
    bat = batch.astype(jnp.int32)

    ntch = TOK // CET
    xf2 = jnp.pad(x, ((0, NPAD - N), (0, 0))).T.reshape(ntch // NW, NW, CET)
    xf2 = xf2.swapaxes(0, 1).reshape(ntch, CET)
    nrep2 = (jnp.arange(TOK, dtype=jnp.int32) % NPAD).reshape(
        ntch // NW, NW, CET).swapaxes(0, 1).reshape(ntch, CET)
    src2 = ei[0].reshape(E // CEM, CEM)
    dst2 = ei[1].reshape(E // CEM, CEM)
    batp = jnp.pad(bat, (0, NPAD - N), constant_values=-1)

    _deg, _scat_tok, _scat_msg = _sc_kernels()
    p = _proj(emb_table, W1)
    deg = _deg(ei[1].reshape(E // CED, CED))
    tokacc = _scat_tok(p, xf2, nrep2)
    dinv16, hw1s = _prep(deg, tokacc)
    parts1 = _scat_msg(hw1s, src2, dst2)
    hw2s = _mm2(parts1, hw1s, dinv16, W2, b1.reshape(1, H))
    parts2 = _scat_msg(hw2s, src2, dst2)
    return _pool(parts2, hw2s, dinv16, batp, b2.reshape(1, H), Wfc,
                 bfc.reshape(1, NCLS))
